# baseline (speedup 1.0000x reference)
import functools

import jax
import jax.numpy as jnp
import numpy as np
from jax import lax
from jax.experimental import layout as jlayout
from jax.experimental import pallas as pl
from jax.experimental.pallas import tpu as pltpu
from jax.experimental.pallas import tpu_sc as plsc

BATCH = 16384
D = 32
OUT_D = 4 * D
NUM_CORES = 2
NUM_SUBCORES = 16
NUM_WORKERS = NUM_CORES * NUM_SUBCORES
BPW = BATCH // NUM_WORKERS
L = 16
GROUPS = BPW // L
OFF_GENDER = 0
OFF_AGE = 2
OFF_OCC = 9
SMALL_ROWS = 30


def _embed_body(g_hbm, a_hbm, o_hbm, z_hbm, small_hbm, wz_hbm, out_hbm,
                gi, ai, oi, zi, small_v, zbuf, big, sem):
    wid = lax.axis_index("s") * NUM_CORES + lax.axis_index("c")
    base = wid * BPW
    pltpu.sync_copy(g_hbm.at[pl.ds(base, BPW)], gi)
    pltpu.sync_copy(a_hbm.at[pl.ds(base, BPW)], ai)
    pltpu.sync_copy(o_hbm.at[pl.ds(base, BPW)], oi)
    pltpu.sync_copy(z_hbm.at[pl.ds(base, BPW)], zi)
    pltpu.sync_copy(small_hbm, small_v)
    cz = pltpu.async_copy(wz_hbm.at[zi], zbuf, sem)

    iota = lax.iota(jnp.int32, L)

    def small_group(g, carry):
        row_off = (g * L + iota) * OUT_D
        for tbl, (buf, off) in enumerate(((gi, OFF_GENDER), (ai, OFF_AGE),
                                          (oi, OFF_OCC))):
            idxv = buf[pl.ds(g * L, L)]
            fb = (idxv + off) * D
            col0 = tbl * D
            for c in range(D):
                v = plsc.load_gather(small_v, [fb + c])
                plsc.store_scatter(big, [row_off + (col0 + c)], v)
        return carry

    lax.fori_loop(0, GROUPS, small_group, 0)

    cz.wait()

    def compact_row(r, carry):
        dst = r * OUT_D + 3 * D
        big[pl.ds(dst, L)] = zbuf[r, pl.ds(0, L)]
        big[pl.ds(dst + L, L)] = zbuf[r, pl.ds(L, L)]
        return carry

    lax.fori_loop(0, BPW, compact_row, 0)

    pltpu.sync_copy(big, out_hbm.at[pl.ds(base * OUT_D, BPW * OUT_D)])


def _embed(gender_idx, age_idx, occupation_idx, area_idx, small_flat, W_area):
    mesh = plsc.VectorSubcoreMesh(core_axis_name="c", subcore_axis_name="s")
    k = functools.partial(
        pl.kernel,
        mesh=mesh,
        out_type=jax.ShapeDtypeStruct((BATCH * OUT_D,), jnp.float32),
        scratch_types=[
            pltpu.VMEM((BPW,), jnp.int32),
            pltpu.VMEM((BPW,), jnp.int32),
            pltpu.VMEM((BPW,), jnp.int32),
            pltpu.VMEM((BPW,), jnp.int32),
            pltpu.VMEM((SMALL_ROWS * D,), jnp.float32),
            pltpu.VMEM((BPW, D), jnp.float32),
            pltpu.VMEM((BPW * OUT_D,), jnp.float32),
            pltpu.SemaphoreType.DMA,
        ],
        compiler_params=pltpu.CompilerParams(use_tc_tiling_on_sc=False,
                                             needs_layout_passes=False),
    )(_embed_body)
    return k(gender_idx, age_idx, occupation_idx, area_idx, small_flat, W_area)


def _retile_body(flat_hbm, out_hbm, buf, sem):
    wid = lax.axis_index("s") * NUM_CORES + lax.axis_index("c")
    base = wid * BPW
    pltpu.sync_copy(flat_hbm.at[pl.ds(base * OUT_D, BPW * OUT_D)], buf)

    def put_row(i, carry):
        pltpu.async_copy(buf.at[pl.ds(i * OUT_D, OUT_D)],
                         out_hbm.at[base + i], sem)
        return carry

    lax.fori_loop(0, BPW, put_row, 0)

    def drain(i, carry):
        pltpu.make_async_copy(buf.at[pl.ds(0, OUT_D)],
                              out_hbm.at[base], sem).wait()
        return carry

    lax.fori_loop(0, BPW, drain, 0)


def _retile(flat):
    mesh = plsc.VectorSubcoreMesh(core_axis_name="c", subcore_axis_name="s")
    k = functools.partial(
        pl.kernel,
        mesh=mesh,
        out_type=jax.ShapeDtypeStruct((BATCH, OUT_D), jnp.float32),
        scratch_types=[
            pltpu.VMEM((BPW * OUT_D,), jnp.float32),
            pltpu.SemaphoreType.DMA,
        ],
        compiler_params=pltpu.CompilerParams(needs_layout_passes=False),
    )(_retile_body)
    return k(flat)


@jax.jit
def _run(gender_idx, age_idx, occupation_idx, area_idx, small_flat, W_area):
    flat = _embed(gender_idx, age_idx, occupation_idx, area_idx,
                  small_flat, W_area)
    return _retile(flat)


def kernel(gender_idx, age_idx, occupation_idx, area_idx,
           W_gender, W_age, W_occupation, W_area):
    small_flat = jnp.concatenate(
        (W_gender, W_age, W_occupation), axis=0).reshape(-1)
    return _run(gender_idx.astype(jnp.int32), age_idx.astype(jnp.int32),
                occupation_idx.astype(jnp.int32), area_idx.astype(jnp.int32),
                small_flat, W_area)


# speedup vs baseline: 1.0212x; 1.0212x over previous
import functools

import jax
import jax.numpy as jnp
import numpy as np
from jax import lax
from jax.experimental import layout as jlayout
from jax.experimental import pallas as pl
from jax.experimental.pallas import tpu as pltpu
from jax.experimental.pallas import tpu_sc as plsc

BATCH = 16384
D = 32
OUT_D = 4 * D
NUM_CORES = 2
NUM_SUBCORES = 16
NUM_WORKERS = NUM_CORES * NUM_SUBCORES
BPW = BATCH // NUM_WORKERS
L = 16
GROUPS = BPW // L
OFF_GENDER = 0
OFF_AGE = 2
OFF_OCC = 9
SMALL_ROWS = 30


def _embed_body(g_hbm, a_hbm, o_hbm, z_hbm, small_hbm, wz_hbm, out_hbm,
                gi, ai, oi, zi, small_v, zbuf, big, sem):
    wid = lax.axis_index("s") * NUM_CORES + lax.axis_index("c")
    base = wid * BPW
    pltpu.sync_copy(g_hbm.at[pl.ds(base, BPW)], gi)
    pltpu.sync_copy(a_hbm.at[pl.ds(base, BPW)], ai)
    pltpu.sync_copy(o_hbm.at[pl.ds(base, BPW)], oi)
    pltpu.sync_copy(z_hbm.at[pl.ds(base, BPW)], zi)
    pltpu.sync_copy(small_hbm, small_v)
    cz = pltpu.async_copy(wz_hbm.at[zi], zbuf, sem)

    iota = lax.iota(jnp.int32, L)

    def small_group(g, carry):
        rows = g * L + iota
        for tbl, (buf, off) in enumerate(((gi, OFF_GENDER), (ai, OFF_AGE),
                                          (oi, OFF_OCC))):
            idxv = buf[pl.ds(g * L, L)]
            fb = (idxv + off) * D
            col0 = tbl * D
            for c in range(D):
                v = plsc.load_gather(small_v, [fb + c])
                plsc.store_scatter(big, [rows, iota * 0 + (col0 + c)], v)
        return carry

    lax.fori_loop(0, GROUPS, small_group, 0)

    cz.wait()

    def compact_row(r, carry):
        big[r, pl.ds(3 * D, L)] = zbuf[r, pl.ds(0, L)]
        big[r, pl.ds(3 * D + L, L)] = zbuf[r, pl.ds(L, L)]
        return carry

    lax.fori_loop(0, BPW, compact_row, 0)

    pltpu.sync_copy(big, out_hbm.at[pl.ds(base, BPW)])


def _embed(gender_idx, age_idx, occupation_idx, area_idx, small_flat, W_area):
    mesh = plsc.VectorSubcoreMesh(core_axis_name="c", subcore_axis_name="s")
    k = functools.partial(
        pl.kernel,
        mesh=mesh,
        out_type=jax.ShapeDtypeStruct((BATCH, OUT_D), jnp.float32),
        scratch_types=[
            pltpu.VMEM((BPW,), jnp.int32),
            pltpu.VMEM((BPW,), jnp.int32),
            pltpu.VMEM((BPW,), jnp.int32),
            pltpu.VMEM((BPW,), jnp.int32),
            pltpu.VMEM((SMALL_ROWS * D,), jnp.float32),
            pltpu.VMEM((BPW, D), jnp.float32),
            pltpu.VMEM((BPW, OUT_D), jnp.float32),
            pltpu.SemaphoreType.DMA,
        ],
        compiler_params=pltpu.CompilerParams(use_tc_tiling_on_sc=False,
                                             needs_layout_passes=False),
    )(_embed_body)
    return k(gender_idx, age_idx, occupation_idx, area_idx, small_flat, W_area)


_embed_jit = jax.jit(_embed)


def kernel(gender_idx, age_idx, occupation_idx, area_idx,
           W_gender, W_age, W_occupation, W_area):
    small_flat = jnp.concatenate(
        (W_gender, W_age, W_occupation), axis=0).reshape(-1)
    return _embed_jit(gender_idx.astype(jnp.int32), age_idx.astype(jnp.int32),
                      occupation_idx.astype(jnp.int32),
                      area_idx.astype(jnp.int32), small_flat, W_area)
